# baseline (speedup 1.0000x reference)
import functools

import jax
import jax.numpy as jnp
from jax import lax
from jax.experimental import pallas as pl
from jax.experimental.pallas import tpu as pltpu
from jax.experimental.pallas import tpu_sc as plsc

BATCH = 4096
HIST = 50
EMBED = 64
VOCAB = 1000000
NC = 2
NS = 16
NW = NC * NS
B = BATCH * HIST
BPW = B // NW
CHUNK = 128
KCH = 2
ROWS_R = KCH * CHUNK
ROUNDS = BPW // ROWS_R
OUT_R = ROWS_R * EMBED


def _emb_body(idx_hbm, table_hbm, out_hbm, idx_v, gidx_v, off_v, rows_v,
              out_v, sem_g, sem_s):
    wid = lax.axis_index("s") * NC + lax.axis_index("c")
    base = wid * BPW
    pltpu.sync_copy(idx_hbm.at[wid], idx_v)

    def pre(i, _):
        v = idx_v[pl.ds(i * 16, 16)]
        gidx_v[pl.ds(i * 16, 16)] = v >> 1
        off_v[pl.ds(i * 16, 16)] = v & 1
        return 0

    lax.fori_loop(0, BPW // 16, pre, 0)

    def fire_gathers(r, buf):
        for k in range(KCH):
            pltpu.async_copy(
                table_hbm.at[gidx_v.at[pl.ds(r * ROWS_R + k * CHUNK, CHUNK)]],
                rows_v.at[buf, pl.ds(k * CHUNK, CHUNK)],
                sem_g.at[buf],
            )

    def drain_gathers(buf):
        for k in range(KCH):
            pltpu.make_async_copy(
                table_hbm.at[gidx_v.at[pl.ds(0, CHUNK)]],
                rows_v.at[buf, pl.ds(k * CHUNK, CHUNK)],
                sem_g.at[buf],
            ).wait()

    def wait_store(buf):
        pltpu.make_async_copy(
            out_v.at[pl.ds(buf * OUT_R, OUT_R)],
            out_hbm.at[pl.ds(0, OUT_R)],
            sem_s.at[buf],
        ).wait()

    fire_gathers(0, 0)

    def round_step(r, buf):
        other = 1 - buf
        drain_gathers(buf)

        @pl.when(r >= 2)
        def _():
            wait_store(buf)

        @pl.when(r + 1 < ROUNDS)
        def _():
            fire_gathers(r + 1, other)

        def compact(g, _):
            goff = off_v[pl.ds(r * ROWS_R + g * 16, 16)]
            for j in range(16):
                spl = goff.at[jnp.full((16,), j, jnp.int32)].get(
                    mode="promise_in_bounds")
                f = spl.astype(jnp.float32)
                row = g * 16 + j
                for k in range(EMBED // 16):
                    left = rows_v[buf, row, pl.ds(k * 16, 16)]
                    right = rows_v[buf, row, pl.ds(64 + k * 16, 16)]
                    out_v[pl.ds((buf * ROWS_R + row) * EMBED + k * 16, 16)] = (
                        left + (right - left) * f
                    )
            return 0

        lax.fori_loop(0, ROWS_R // 16, compact, 0)

        pltpu.async_copy(
            out_v.at[pl.ds(buf * OUT_R, OUT_R)],
            out_hbm.at[pl.ds((base + r * ROWS_R) * EMBED, OUT_R)],
            sem_s.at[buf],
        )

    def body(i, _):
        round_step(2 * i, 0)
        round_step(2 * i + 1, 1)
        return 0

    lax.fori_loop(0, ROUNDS // 2, body, 0)
    round_step(ROUNDS - 1, 0)

    wait_store(0)
    wait_store(1)


@jax.jit
def _emb(idx, table2):
    mesh = plsc.VectorSubcoreMesh(core_axis_name="c", subcore_axis_name="s")
    f = functools.partial(
        pl.kernel,
        mesh=mesh,
        out_type=jax.ShapeDtypeStruct((B * EMBED,), jnp.float32),
        scratch_types=[
            pltpu.VMEM((BPW,), jnp.int32),
            pltpu.VMEM((BPW,), jnp.int32),
            pltpu.VMEM((BPW,), jnp.int32),
            pltpu.VMEM((2, ROWS_R, 128), jnp.float32),
            pltpu.VMEM((2 * OUT_R,), jnp.float32),
            pltpu.SemaphoreType.DMA((2,)),
            pltpu.SemaphoreType.DMA((2,)),
        ],
    )(_emb_body)
    return f(idx, table2)


def kernel(indices, table):
    idx = indices.reshape(NW, BPW)
    table2 = table.reshape(VOCAB // 2, 2 * EMBED)
    out = _emb(idx, table2)
    return out.reshape(BATCH, HIST, EMBED)
